# baseline (speedup 1.0000x reference)
import functools

import jax
import jax.numpy as jnp
from jax import lax
from jax.experimental import pallas as pl
from jax.experimental.pallas import tpu as pltpu
from jax.experimental.pallas import tpu_sc as plsc

B = 8
N = 10000
C = 2
E = 160000
D = 256
H = 128
NTILE = 16
EPT = C * E // NTILE
K = 80
NCHUNK = EPT // K
PD = 3
BLK = 2000
NBLK = EPT // BLK
CPB = BLK // K
NP = 10240
RPT = NP // NTILE
RC = 80


def _build_sc_call():
    mesh = plsc.VectorSubcoreMesh(core_axis_name="c", subcore_axis_name="s")

    @functools.partial(
        pl.kernel,
        mesh=mesh,
        out_type=jax.ShapeDtypeStruct((B, N, D), jnp.float32),
        scratch_types=[
            pltpu.VMEM_SHARED((NP, H), jnp.float32),
            pltpu.VMEM((2 * BLK,), jnp.int32),
            pltpu.VMEM((2 * BLK,), jnp.int32),
            pltpu.VMEM((2 * BLK,), jnp.float32),
            pltpu.VMEM((K,), jnp.int32),
            pltpu.VMEM((K,), jnp.int32),
            pltpu.VMEM((K,), jnp.int32),
            pltpu.VMEM((K,), jnp.int32),
            pltpu.VMEM((K,), jnp.int32),
            pltpu.VMEM((K,), jnp.int32),
            pltpu.VMEM((K, H), jnp.float32),
            pltpu.VMEM((K, H), jnp.float32),
            pltpu.VMEM((K, H), jnp.float32),
            pltpu.VMEM((16,), jnp.float32),
            pltpu.SemaphoreType.DMA,
            pltpu.SemaphoreType.DMA,
            pltpu.SemaphoreType.DMA,
            pltpu.SemaphoreType.DMA,
            pltpu.SemaphoreType.DMA,
            pltpu.SemaphoreType.DMA,
            pltpu.SemaphoreType.DMA,
        ],
    )
    def gin_sc(xt_hbm, rows_hbm, cols_hbm, vals_hbm, esv_hbm, out_hbm,
               acc, rows_st, cols_st, vals_st,
               idxv0, idxv1, idxv2, rowv0, rowv1, rowv2,
               buf0, buf1, buf2, epsv,
               sem_g0, sem_g1, sem_g2, sem_s, sem_r, sem_c, sem_v):
        h = lax.axis_index("c")
        s = lax.axis_index("s")
        pltpu.sync_copy(esv_hbm, epsv)
        esplat = epsv[pl.ds(0, 16)]
        r0 = s * RPT
        nrch = jnp.minimum(RPT, jnp.maximum(0, N - r0)) // RC
        idxvs = (idxv0, idxv1, idxv2)
        rowvs = (rowv0, rowv1, rowv2)
        bufs = (buf0, buf1, buf2)
        sems = (sem_g0, sem_g1, sem_g2)

        ho = pl.multiple_of(h * H, 128)

        def per_graph(b, carry):
            hbase = b * N
            eoff = b * (C * E) + s * EPT

            def stage_start(blk):
                po = pl.multiple_of((blk % 2) * BLK, 8)
                off = eoff + blk * BLK
                pltpu.async_copy(rows_hbm.at[pl.ds(off, BLK)],
                                 rows_st.at[pl.ds(po, BLK)], sem_r)
                pltpu.async_copy(cols_hbm.at[pl.ds(off, BLK)],
                                 cols_st.at[pl.ds(po, BLK)], sem_c)
                pltpu.async_copy(vals_hbm.at[pl.ds(off, BLK)],
                                 vals_st.at[pl.ds(po, BLK)], sem_v)

            def stage_wait(blk):
                po = pl.multiple_of((blk % 2) * BLK, 8)
                off = eoff + blk * BLK
                pltpu.make_async_copy(rows_hbm.at[pl.ds(off, BLK)],
                                      rows_st.at[pl.ds(po, BLK)], sem_r).wait()
                pltpu.make_async_copy(cols_hbm.at[pl.ds(off, BLK)],
                                      cols_st.at[pl.ds(po, BLK)], sem_c).wait()
                pltpu.make_async_copy(vals_hbm.at[pl.ds(off, BLK)],
                                      vals_st.at[pl.ds(po, BLK)], sem_v).wait()

            def build_idx(g, dstv):
                p = (g // CPB) % 2
                gb = g % CPB
                for j in range(K // 16):
                    src = pl.multiple_of(p * BLK + gb * K + j * 16, 16)
                    dstv[pl.ds(j * 16, 16)] = cols_st[pl.ds(src, 16)] + hbase

            stage_start(0)

            def init_chunk(rc, c2):
                ro = r0 + rc * RC
                pltpu.sync_copy(
                    xt_hbm.at[pl.ds(hbase + ro, RC), pl.ds(ho, H)], buf0)

                @plsc.parallel_loop(0, RC, unroll=2)
                def _(i):
                    for j in range(H // 16):
                        sl = pl.ds(j * 16, 16)
                        buf0[i, sl] = esplat * buf0[i, sl]

                pltpu.sync_copy(buf0, acc.at[pl.ds(ro, RC)])
                return c2

            lax.fori_loop(0, nrch, init_chunk, 0)

            stage_wait(0)
            stage_start(1)
            plsc.subcore_barrier()

            def scale_chunk(g, buf):
                p = (g // CPB) % 2
                gb = g % CPB

                @plsc.parallel_loop(0, K, unroll=4)
                def _(i):
                    ga = i // 16
                    i2 = i - ga * 16
                    voff = pl.multiple_of(p * BLK + gb * K + ga * 16, 16)
                    vvec = vals_st[pl.ds(voff, 16)]
                    vsplat = vvec[jnp.full((16,), i2, jnp.int32)]
                    for j in range(H // 16):
                        sl = pl.ds(j * 16, 16)
                        buf[i, sl] = vsplat * buf[i, sl]

            def process(g, par):
                pb = (par + 2) % PD

                @pl.when((g % CPB == 0) & (g >= CPB)
                         & (g <= (NBLK - 2) * CPB))
                def _():
                    stage_start(g // CPB + 1)

                @pl.when(g + 2 < NCHUNK)
                def _():
                    @pl.when(g >= 1)
                    def _():
                        pltpu.make_async_copy(
                            bufs[pb], acc.at[rowvs[pb]], sem_s).wait()

                    @pl.when((g + 2) % CPB == 0)
                    def _():
                        stage_wait((g + 2) // CPB)

                    build_idx(g + 2, idxvs[pb])
                    pltpu.async_copy(xt_hbm.at[idxvs[pb], pl.ds(ho, H)],
                                     bufs[pb], sems[pb])

                p = (g // CPB) % 2
                gb = g % CPB
                for j in range(K // 16):
                    src = pl.multiple_of(p * BLK + gb * K + j * 16, 16)
                    rowvs[par][pl.ds(j * 16, 16)] = rows_st[pl.ds(src, 16)]
                pltpu.make_async_copy(xt_hbm.at[idxvs[par], pl.ds(ho, H)],
                                      bufs[par], sems[par]).wait()
                scale_chunk(g, bufs[par])
                pltpu.async_copy(bufs[par], acc.at[rowvs[par]], sem_s,
                                 add=True)

            build_idx(0, idxvs[0])
            pltpu.async_copy(xt_hbm.at[idxvs[0], pl.ds(ho, H)], bufs[0],
                             sems[0])
            build_idx(1, idxvs[1])
            pltpu.async_copy(xt_hbm.at[idxvs[1], pl.ds(ho, H)], bufs[1],
                             sems[1])

            def outer(g3, c2):
                for par in range(PD):
                    process(g3 * PD + par, par)
                return c2

            lax.fori_loop(0, (NCHUNK - 1) // PD, outer, 0)
            process(jnp.int32(NCHUNK - 1), (NCHUNK - 1) % PD)

            for q in (NCHUNK - 3, NCHUNK - 2, NCHUNK - 1):
                pltpu.make_async_copy(bufs[q % PD], acc.at[rowvs[q % PD]],
                                      sem_s).wait()
            plsc.subcore_barrier()

            def out_chunk(rc, c2):
                ro = r0 + rc * RC
                pltpu.sync_copy(acc.at[pl.ds(ro, RC)], buf0)
                pltpu.sync_copy(buf0,
                                out_hbm.at[b, pl.ds(ro, RC), pl.ds(ho, H)])
                return c2

            lax.fori_loop(0, nrch, out_chunk, 0)
            plsc.subcore_barrier()
            return carry

        lax.fori_loop(0, B, per_graph, 0)

    return gin_sc


_GIN_SC = _build_sc_call()


@jax.jit
def _gin(inputs, edge_index, adj_values, epsilon):
    ei = edge_index.astype(jnp.int32)
    rows = ei[:, :, 0, :].reshape(B * C * E)
    cols = ei[:, :, 1, :].reshape(B * C * E)
    vals = adj_values.astype(jnp.float32).reshape(B * C * E)
    xt = inputs.reshape(B * N, D)
    esv = jnp.full((16,), jnp.sum(epsilon), jnp.float32)
    return _GIN_SC(xt, rows, cols, vals, esv)


def kernel(inputs, edge_index, adj_values, epsilon):
    return _gin(inputs, edge_index, adj_values, epsilon)


# speedup vs baseline: 1.1532x; 1.1532x over previous
import functools

import jax
import jax.numpy as jnp
from jax import lax
from jax.experimental import pallas as pl
from jax.experimental.pallas import tpu as pltpu
from jax.experimental.pallas import tpu_sc as plsc

B = 8
N = 10000
C = 2
E = 160000
D = 256
H = 128
NTILE = 16
EPT = C * E // NTILE
K = 80
NCHUNK = EPT // K
PD = 3
BLK = 2000
NBLK = EPT // BLK
CPB = BLK // K
NP = 10240
RPT = NP // NTILE
RC = 80


def _build_sc_call():
    mesh = plsc.VectorSubcoreMesh(core_axis_name="c", subcore_axis_name="s")

    @functools.partial(
        pl.kernel,
        mesh=mesh,
        out_type=jax.ShapeDtypeStruct((B, N, D), jnp.float32),
        scratch_types=[
            pltpu.VMEM_SHARED((NP, H), jnp.float32),
            pltpu.VMEM((2 * BLK,), jnp.int32),
            pltpu.VMEM((2 * BLK,), jnp.int32),
            pltpu.VMEM((2 * BLK,), jnp.float32),
            pltpu.VMEM((K,), jnp.int32),
            pltpu.VMEM((K,), jnp.int32),
            pltpu.VMEM((K,), jnp.int32),
            pltpu.VMEM((K,), jnp.int32),
            pltpu.VMEM((K,), jnp.int32),
            pltpu.VMEM((K,), jnp.int32),
            pltpu.VMEM((K, H), jnp.float32),
            pltpu.VMEM((K, H), jnp.float32),
            pltpu.VMEM((K, H), jnp.float32),
            pltpu.VMEM((16,), jnp.float32),
            pltpu.SemaphoreType.DMA,
            pltpu.SemaphoreType.DMA,
            pltpu.SemaphoreType.DMA,
            pltpu.SemaphoreType.DMA,
            pltpu.SemaphoreType.DMA,
            pltpu.SemaphoreType.DMA,
            pltpu.SemaphoreType.DMA,
        ],
    )
    def gin_sc(xt_hbm, rows_hbm, cols_hbm, vals_hbm, esv_hbm, out_hbm,
               acc, rows_st, cols_st, vals_st,
               idxv0, idxv1, idxv2, rowv0, rowv1, rowv2,
               buf0, buf1, buf2, epsv,
               sem_g0, sem_g1, sem_g2, sem_s, sem_r, sem_c, sem_v):
        h = lax.axis_index("c")
        s = lax.axis_index("s")
        pltpu.sync_copy(esv_hbm, epsv)
        esplat = epsv[pl.ds(0, 16)]
        r0 = s * RPT
        nrch = jnp.minimum(RPT, jnp.maximum(0, N - r0)) // RC
        idxvs = (idxv0, idxv1, idxv2)
        rowvs = (rowv0, rowv1, rowv2)
        bufs = (buf0, buf1, buf2)
        sems = (sem_g0, sem_g1, sem_g2)

        ho = pl.multiple_of(h * H, 128)

        def per_graph(b, carry):
            hbase = (b * 2 + h) * N
            eoff = b * (C * E) + s * EPT

            def stage_start(blk):
                po = pl.multiple_of((blk % 2) * BLK, 8)
                off = eoff + blk * BLK
                pltpu.async_copy(rows_hbm.at[pl.ds(off, BLK)],
                                 rows_st.at[pl.ds(po, BLK)], sem_r)
                pltpu.async_copy(cols_hbm.at[pl.ds(off, BLK)],
                                 cols_st.at[pl.ds(po, BLK)], sem_c)
                pltpu.async_copy(vals_hbm.at[pl.ds(off, BLK)],
                                 vals_st.at[pl.ds(po, BLK)], sem_v)

            def stage_wait(blk):
                po = pl.multiple_of((blk % 2) * BLK, 8)
                off = eoff + blk * BLK
                pltpu.make_async_copy(rows_hbm.at[pl.ds(off, BLK)],
                                      rows_st.at[pl.ds(po, BLK)], sem_r).wait()
                pltpu.make_async_copy(cols_hbm.at[pl.ds(off, BLK)],
                                      cols_st.at[pl.ds(po, BLK)], sem_c).wait()
                pltpu.make_async_copy(vals_hbm.at[pl.ds(off, BLK)],
                                      vals_st.at[pl.ds(po, BLK)], sem_v).wait()

            def build_idx(g, dstv):
                p = (g // CPB) % 2
                gb = g % CPB
                for j in range(K // 16):
                    src = pl.multiple_of(p * BLK + gb * K + j * 16, 16)
                    dstv[pl.ds(j * 16, 16)] = cols_st[pl.ds(src, 16)] + hbase

            stage_start(0)

            def init_chunk(rc, c2):
                ro = r0 + rc * RC
                pltpu.sync_copy(xt_hbm.at[pl.ds(hbase + ro, RC)], buf0)

                @plsc.parallel_loop(0, RC, unroll=2)
                def _(i):
                    for j in range(H // 16):
                        sl = pl.ds(j * 16, 16)
                        buf0[i, sl] = esplat * buf0[i, sl]

                pltpu.sync_copy(buf0, acc.at[pl.ds(ro, RC)])
                return c2

            lax.fori_loop(0, nrch, init_chunk, 0)

            stage_wait(0)
            stage_start(1)
            plsc.subcore_barrier()

            def scale_chunk(g, buf):
                p = (g // CPB) % 2
                gb = g % CPB

                @plsc.parallel_loop(0, K, unroll=4)
                def _(i):
                    ga = i // 16
                    i2 = i - ga * 16
                    voff = pl.multiple_of(p * BLK + gb * K + ga * 16, 16)
                    vvec = vals_st[pl.ds(voff, 16)]
                    vsplat = vvec[jnp.full((16,), i2, jnp.int32)]
                    for j in range(H // 16):
                        sl = pl.ds(j * 16, 16)
                        buf[i, sl] = vsplat * buf[i, sl]

            def process(g, par):
                pb = (par + 2) % PD

                @pl.when((g % CPB == 0) & (g >= CPB)
                         & (g <= (NBLK - 2) * CPB))
                def _():
                    stage_start(g // CPB + 1)

                @pl.when(g + 2 < NCHUNK)
                def _():
                    @pl.when(g >= 1)
                    def _():
                        pltpu.make_async_copy(
                            bufs[pb], acc.at[rowvs[pb]], sem_s).wait()

                    @pl.when((g + 2) % CPB == 0)
                    def _():
                        stage_wait((g + 2) // CPB)

                    build_idx(g + 2, idxvs[pb])
                    pltpu.async_copy(xt_hbm.at[idxvs[pb]], bufs[pb],
                                     sems[pb])

                p = (g // CPB) % 2
                gb = g % CPB
                for j in range(K // 16):
                    src = pl.multiple_of(p * BLK + gb * K + j * 16, 16)
                    rowvs[par][pl.ds(j * 16, 16)] = rows_st[pl.ds(src, 16)]
                pltpu.make_async_copy(xt_hbm.at[idxvs[par]], bufs[par],
                                      sems[par]).wait()
                scale_chunk(g, bufs[par])
                pltpu.async_copy(bufs[par], acc.at[rowvs[par]], sem_s,
                                 add=True)

            build_idx(0, idxvs[0])
            pltpu.async_copy(xt_hbm.at[idxvs[0]], bufs[0], sems[0])
            build_idx(1, idxvs[1])
            pltpu.async_copy(xt_hbm.at[idxvs[1]], bufs[1], sems[1])

            def outer(g3, c2):
                for par in range(PD):
                    process(g3 * PD + par, par)
                return c2

            lax.fori_loop(0, (NCHUNK - 1) // PD, outer, 0)
            process(jnp.int32(NCHUNK - 1), (NCHUNK - 1) % PD)

            for q in (NCHUNK - 3, NCHUNK - 2, NCHUNK - 1):
                pltpu.make_async_copy(bufs[q % PD], acc.at[rowvs[q % PD]],
                                      sem_s).wait()
            plsc.subcore_barrier()

            def out_chunk(rc, c2):
                ro = r0 + rc * RC
                pltpu.sync_copy(acc.at[pl.ds(ro, RC)], buf0)
                pltpu.sync_copy(buf0,
                                out_hbm.at[b, pl.ds(ro, RC), pl.ds(ho, H)])
                return c2

            lax.fori_loop(0, nrch, out_chunk, 0)
            plsc.subcore_barrier()
            return carry

        lax.fori_loop(0, B, per_graph, 0)

    return gin_sc


_GIN_SC = _build_sc_call()


@jax.jit
def _gin(inputs, edge_index, adj_values, epsilon):
    ei = edge_index.astype(jnp.int32)
    rows = ei[:, :, 0, :].reshape(B * C * E)
    cols = ei[:, :, 1, :].reshape(B * C * E)
    vals = adj_values.astype(jnp.float32).reshape(B * C * E)
    xt = (inputs.reshape(B, N, 2, H)
          .transpose(0, 2, 1, 3)
          .reshape(B * 2 * N, H))
    esv = jnp.full((16,), jnp.sum(epsilon), jnp.float32)
    return _GIN_SC(xt, rows, cols, vals, esv)


def kernel(inputs, edge_index, adj_values, epsilon):
    return _gin(inputs, edge_index, adj_values, epsilon)


# speedup vs baseline: 1.1604x; 1.0063x over previous
import functools

import jax
import jax.numpy as jnp
from jax import lax
from jax.experimental import pallas as pl
from jax.experimental.pallas import tpu as pltpu
from jax.experimental.pallas import tpu_sc as plsc

B = 8
N = 10000
C = 2
E = 160000
D = 256
H = 128
NTILE = 16
EPT = C * E // NTILE
K = 80
NCHUNK = EPT // K
PD = 4
BLK = 800
NBLK = EPT // BLK
CPB = BLK // K
NP = 10240
RPT = NP // NTILE
RC = 80


def _build_sc_call():
    mesh = plsc.VectorSubcoreMesh(core_axis_name="c", subcore_axis_name="s")

    @functools.partial(
        pl.kernel,
        mesh=mesh,
        out_type=jax.ShapeDtypeStruct((B, N, D), jnp.float32),
        scratch_types=[
            pltpu.VMEM_SHARED((NP, H), jnp.float32),
            pltpu.VMEM((2 * BLK,), jnp.int32),
            pltpu.VMEM((2 * BLK,), jnp.int32),
            pltpu.VMEM((2 * BLK,), jnp.float32),
            pltpu.VMEM((K,), jnp.int32),
            pltpu.VMEM((K,), jnp.int32),
            pltpu.VMEM((K,), jnp.int32),
            pltpu.VMEM((K,), jnp.int32),
            pltpu.VMEM((K,), jnp.int32),
            pltpu.VMEM((K,), jnp.int32),
            pltpu.VMEM((K,), jnp.int32),
            pltpu.VMEM((K,), jnp.int32),
            pltpu.VMEM((K, H), jnp.float32),
            pltpu.VMEM((K, H), jnp.float32),
            pltpu.VMEM((K, H), jnp.float32),
            pltpu.VMEM((K, H), jnp.float32),
            pltpu.VMEM((16,), jnp.float32),
            pltpu.SemaphoreType.DMA,
            pltpu.SemaphoreType.DMA,
            pltpu.SemaphoreType.DMA,
            pltpu.SemaphoreType.DMA,
            pltpu.SemaphoreType.DMA,
            pltpu.SemaphoreType.DMA,
            pltpu.SemaphoreType.DMA,
            pltpu.SemaphoreType.DMA,
        ],
    )
    def gin_sc(xt_hbm, rows_hbm, cols_hbm, vals_hbm, esv_hbm, out_hbm,
               acc, rows_st, cols_st, vals_st,
               idxv0, idxv1, idxv2, idxv3, rowv0, rowv1, rowv2, rowv3,
               buf0, buf1, buf2, buf3, epsv,
               sem_g0, sem_g1, sem_g2, sem_g3, sem_s, sem_r, sem_c, sem_v):
        h = lax.axis_index("c")
        s = lax.axis_index("s")
        pltpu.sync_copy(esv_hbm, epsv)
        esplat = epsv[pl.ds(0, 16)]
        r0 = s * RPT
        nrch = jnp.minimum(RPT, jnp.maximum(0, N - r0)) // RC
        idxvs = (idxv0, idxv1, idxv2, idxv3)
        rowvs = (rowv0, rowv1, rowv2, rowv3)
        bufs = (buf0, buf1, buf2, buf3)
        sems = (sem_g0, sem_g1, sem_g2, sem_g3)

        ho = pl.multiple_of(h * H, 128)

        def per_graph(b, carry):
            hbase = (b * 2 + h) * N
            eoff = b * (C * E) + s * EPT

            def stage_start(blk):
                po = pl.multiple_of((blk % 2) * BLK, 8)
                off = eoff + blk * BLK
                pltpu.async_copy(rows_hbm.at[pl.ds(off, BLK)],
                                 rows_st.at[pl.ds(po, BLK)], sem_r)
                pltpu.async_copy(cols_hbm.at[pl.ds(off, BLK)],
                                 cols_st.at[pl.ds(po, BLK)], sem_c)
                pltpu.async_copy(vals_hbm.at[pl.ds(off, BLK)],
                                 vals_st.at[pl.ds(po, BLK)], sem_v)

            def stage_wait(blk):
                po = pl.multiple_of((blk % 2) * BLK, 8)
                off = eoff + blk * BLK
                pltpu.make_async_copy(rows_hbm.at[pl.ds(off, BLK)],
                                      rows_st.at[pl.ds(po, BLK)], sem_r).wait()
                pltpu.make_async_copy(cols_hbm.at[pl.ds(off, BLK)],
                                      cols_st.at[pl.ds(po, BLK)], sem_c).wait()
                pltpu.make_async_copy(vals_hbm.at[pl.ds(off, BLK)],
                                      vals_st.at[pl.ds(po, BLK)], sem_v).wait()

            def build_idx(g, dstv):
                p = (g // CPB) % 2
                gb = g % CPB
                for j in range(K // 16):
                    src = pl.multiple_of(p * BLK + gb * K + j * 16, 16)
                    dstv[pl.ds(j * 16, 16)] = cols_st[pl.ds(src, 16)] + hbase

            stage_start(0)

            def init_chunk(rc, c2):
                ro = r0 + rc * RC
                pltpu.sync_copy(xt_hbm.at[pl.ds(hbase + ro, RC)], buf0)

                @plsc.parallel_loop(0, RC, unroll=2)
                def _(i):
                    for j in range(H // 16):
                        sl = pl.ds(j * 16, 16)
                        buf0[i, sl] = esplat * buf0[i, sl]

                pltpu.sync_copy(buf0, acc.at[pl.ds(ro, RC)])
                return c2

            lax.fori_loop(0, nrch, init_chunk, 0)

            stage_wait(0)
            stage_start(1)
            plsc.subcore_barrier()

            def scale_chunk(g, buf):
                p = (g // CPB) % 2
                gb = g % CPB

                @plsc.parallel_loop(0, K, unroll=8)
                def _(i):
                    ga = i // 16
                    i2 = i - ga * 16
                    voff = pl.multiple_of(p * BLK + gb * K + ga * 16, 16)
                    vvec = vals_st[pl.ds(voff, 16)]
                    vsplat = vvec[jnp.full((16,), i2, jnp.int32)]
                    for j in range(H // 16):
                        sl = pl.ds(j * 16, 16)
                        buf[i, sl] = vsplat * buf[i, sl]

            def process(g, par):
                pb = (par + 3) % PD

                @pl.when((g % CPB == 0) & (g >= CPB)
                         & (g <= (NBLK - 2) * CPB))
                def _():
                    stage_start(g // CPB + 1)

                @pl.when(g + 3 < NCHUNK)
                def _():
                    @pl.when(g >= 1)
                    def _():
                        pltpu.make_async_copy(
                            bufs[pb], acc.at[rowvs[pb]], sem_s).wait()

                    @pl.when((g + 3) % CPB == 0)
                    def _():
                        stage_wait((g + 3) // CPB)

                    build_idx(g + 3, idxvs[pb])
                    pltpu.async_copy(xt_hbm.at[idxvs[pb]], bufs[pb],
                                     sems[pb])

                p = (g // CPB) % 2
                gb = g % CPB
                for j in range(K // 16):
                    src = pl.multiple_of(p * BLK + gb * K + j * 16, 16)
                    rowvs[par][pl.ds(j * 16, 16)] = rows_st[pl.ds(src, 16)]
                pltpu.make_async_copy(xt_hbm.at[idxvs[par]], bufs[par],
                                      sems[par]).wait()
                scale_chunk(g, bufs[par])
                pltpu.async_copy(bufs[par], acc.at[rowvs[par]], sem_s,
                                 add=True)

            for q in range(PD - 1):
                build_idx(q, idxvs[q])
                pltpu.async_copy(xt_hbm.at[idxvs[q]], bufs[q], sems[q])

            def outer(g4, c2):
                for par in range(PD):
                    process(g4 * PD + par, par)
                return c2

            lax.fori_loop(0, (NCHUNK - 2) // PD, outer, 0)
            for q in range(((NCHUNK - 2) // PD) * PD, NCHUNK):
                process(jnp.int32(q), q % PD)

            for q in range(NCHUNK - PD, NCHUNK):
                pltpu.make_async_copy(bufs[q % PD], acc.at[rowvs[q % PD]],
                                      sem_s).wait()
            plsc.subcore_barrier()

            def out_chunk(rc, c2):
                ro = r0 + rc * RC
                pltpu.sync_copy(acc.at[pl.ds(ro, RC)], buf0)
                pltpu.sync_copy(buf0,
                                out_hbm.at[b, pl.ds(ro, RC), pl.ds(ho, H)])
                return c2

            lax.fori_loop(0, nrch, out_chunk, 0)
            return carry

        lax.fori_loop(0, B, per_graph, 0)

    return gin_sc


_GIN_SC = _build_sc_call()


@jax.jit
def _gin(inputs, edge_index, adj_values, epsilon):
    ei = edge_index.astype(jnp.int32)
    rows = ei[:, :, 0, :].reshape(B * C * E)
    cols = ei[:, :, 1, :].reshape(B * C * E)
    vals = adj_values.astype(jnp.float32).reshape(B * C * E)
    xt = (inputs.reshape(B, N, 2, H)
          .transpose(0, 2, 1, 3)
          .reshape(B * 2 * N, H))
    esv = jnp.full((16,), jnp.sum(epsilon), jnp.float32)
    return _GIN_SC(xt, rows, cols, vals, esv)


def kernel(inputs, edge_index, adj_values, epsilon):
    return _gin(inputs, edge_index, adj_values, epsilon)


# speedup vs baseline: 1.1891x; 1.0247x over previous
import functools

import jax
import jax.numpy as jnp
from jax import lax
from jax.experimental import pallas as pl
from jax.experimental.pallas import tpu as pltpu
from jax.experimental.pallas import tpu_sc as plsc

B = 8
N = 10000
C = 2
E = 160000
D = 256
H = 128
NTILE = 16
EPT = C * E // NTILE
K = 80
NCHUNK = EPT // K
PD = 4
BLK = 800
NBLK = EPT // BLK
CPB = BLK // K
NP = 10240
RPT = NP // NTILE
RC = 80


def _build_sc_call():
    mesh = plsc.VectorSubcoreMesh(core_axis_name="c", subcore_axis_name="s")

    @functools.partial(
        pl.kernel,
        mesh=mesh,
        out_type=jax.ShapeDtypeStruct((B, N, D), jnp.float32),
        scratch_types=[
            pltpu.VMEM_SHARED((NP, H), jnp.float32),
            pltpu.VMEM((2 * BLK,), jnp.int32),
            pltpu.VMEM((2 * BLK,), jnp.int32),
            pltpu.VMEM((2 * BLK,), jnp.float32),
            pltpu.VMEM((K,), jnp.int32),
            pltpu.VMEM((K,), jnp.int32),
            pltpu.VMEM((K,), jnp.int32),
            pltpu.VMEM((K,), jnp.int32),
            pltpu.VMEM((K,), jnp.int32),
            pltpu.VMEM((K,), jnp.int32),
            pltpu.VMEM((K,), jnp.int32),
            pltpu.VMEM((K,), jnp.int32),
            pltpu.VMEM((K, H), jnp.float32),
            pltpu.VMEM((K, H), jnp.float32),
            pltpu.VMEM((K, H), jnp.float32),
            pltpu.VMEM((K, H), jnp.float32),
            pltpu.VMEM((16,), jnp.float32),
            pltpu.SemaphoreType.DMA,
            pltpu.SemaphoreType.DMA,
            pltpu.SemaphoreType.DMA,
            pltpu.SemaphoreType.DMA,
            pltpu.SemaphoreType.DMA,
            pltpu.SemaphoreType.DMA,
            pltpu.SemaphoreType.DMA,
            pltpu.SemaphoreType.DMA,
        ],
    )
    def gin_sc(xt_hbm, rows_hbm, cols_hbm, vals_hbm, esv_hbm, out_hbm,
               acc, rows_st, cols_st, vals_st,
               idxv0, idxv1, idxv2, idxv3, rowv0, rowv1, rowv2, rowv3,
               buf0, buf1, buf2, buf3, epsv,
               sem_g0, sem_g1, sem_g2, sem_g3, sem_s, sem_r, sem_c, sem_v):
        h = lax.axis_index("c")
        s = lax.axis_index("s")
        pltpu.sync_copy(esv_hbm, epsv)
        esplat = epsv[pl.ds(0, 16)]
        r0 = s * RPT
        nrch = jnp.minimum(RPT, jnp.maximum(0, N - r0)) // RC
        idxvs = (idxv0, idxv1, idxv2, idxv3)
        rowvs = (rowv0, rowv1, rowv2, rowv3)
        bufs = (buf0, buf1, buf2, buf3)
        sems = (sem_g0, sem_g1, sem_g2, sem_g3)

        ho = pl.multiple_of(h * H, 128)

        def per_graph(b, carry):
            hbase = (b * 2 + h) * N
            eoff = b * (C * E) + s * EPT

            def stage_start(blk):
                po = pl.multiple_of((blk % 2) * BLK, 8)
                off = eoff + blk * BLK
                pltpu.async_copy(rows_hbm.at[pl.ds(off, BLK)],
                                 rows_st.at[pl.ds(po, BLK)], sem_r)
                pltpu.async_copy(cols_hbm.at[pl.ds(off, BLK)],
                                 cols_st.at[pl.ds(po, BLK)], sem_c)
                pltpu.async_copy(vals_hbm.at[pl.ds(off, BLK)],
                                 vals_st.at[pl.ds(po, BLK)], sem_v)

            def stage_wait(blk):
                po = pl.multiple_of((blk % 2) * BLK, 8)
                off = eoff + blk * BLK
                pltpu.make_async_copy(rows_hbm.at[pl.ds(off, BLK)],
                                      rows_st.at[pl.ds(po, BLK)], sem_r).wait()
                pltpu.make_async_copy(cols_hbm.at[pl.ds(off, BLK)],
                                      cols_st.at[pl.ds(po, BLK)], sem_c).wait()
                pltpu.make_async_copy(vals_hbm.at[pl.ds(off, BLK)],
                                      vals_st.at[pl.ds(po, BLK)], sem_v).wait()

            def build_idx(g, dstv):
                p = (g // CPB) % 2
                gb = g % CPB
                for j in range(K // 16):
                    src = pl.multiple_of(p * BLK + gb * K + j * 16, 16)
                    dstv[pl.ds(j * 16, 16)] = cols_st[pl.ds(src, 16)] + hbase

            stage_start(0)

            MAXRCH = RPT // RC

            def ird(rc):
                ro = r0 + rc * RC
                return (xt_hbm.at[pl.ds(hbase + ro, RC)], bufs[rc % 2],
                        sems[rc % 2])

            def iwr(rc):
                ro = r0 + rc * RC
                return (bufs[rc % 2], acc.at[pl.ds(ro, RC)], sem_s)

            pltpu.async_copy(*ird(0))
            for rc in range(MAXRCH):
                @pl.when(rc < nrch)
                def _(rc=rc):
                    if rc + 1 < MAXRCH:
                        @pl.when(rc + 1 < nrch)
                        def _(rc=rc):
                            if rc >= 1:
                                pltpu.make_async_copy(*iwr(rc - 1)).wait()
                            pltpu.async_copy(*ird(rc + 1))

                    pltpu.make_async_copy(*ird(rc)).wait()
                    bb = bufs[rc % 2]

                    @plsc.parallel_loop(0, RC, unroll=2)
                    def _(i):
                        for j in range(H // 16):
                            sl = pl.ds(j * 16, 16)
                            bb[i, sl] = esplat * bb[i, sl]

                    pltpu.async_copy(*iwr(rc))
            pltpu.make_async_copy(bufs[0], acc.at[pl.ds(r0, RC)],
                                  sem_s).wait()
            pltpu.make_async_copy(bufs[1], acc.at[pl.ds(r0, RC)],
                                  sem_s).wait()

            stage_wait(0)
            stage_start(1)
            plsc.subcore_barrier()

            def scale_chunk(g, buf):
                p = (g // CPB) % 2
                gb = g % CPB

                @plsc.parallel_loop(0, K, unroll=8)
                def _(i):
                    ga = i // 16
                    i2 = i - ga * 16
                    voff = pl.multiple_of(p * BLK + gb * K + ga * 16, 16)
                    vvec = vals_st[pl.ds(voff, 16)]
                    vsplat = vvec[jnp.full((16,), i2, jnp.int32)]
                    for j in range(H // 16):
                        sl = pl.ds(j * 16, 16)
                        buf[i, sl] = vsplat * buf[i, sl]

            def process(g, par):
                pb = (par + 3) % PD

                @pl.when((g % CPB == 0) & (g >= CPB)
                         & (g <= (NBLK - 2) * CPB))
                def _():
                    stage_start(g // CPB + 1)

                @pl.when(g + 3 < NCHUNK)
                def _():
                    @pl.when(g >= 1)
                    def _():
                        pltpu.make_async_copy(
                            bufs[pb], acc.at[rowvs[pb]], sem_s).wait()

                    @pl.when((g + 3) % CPB == 0)
                    def _():
                        stage_wait((g + 3) // CPB)

                    build_idx(g + 3, idxvs[pb])
                    pltpu.async_copy(xt_hbm.at[idxvs[pb]], bufs[pb],
                                     sems[pb])

                p = (g // CPB) % 2
                gb = g % CPB
                for j in range(K // 16):
                    src = pl.multiple_of(p * BLK + gb * K + j * 16, 16)
                    rowvs[par][pl.ds(j * 16, 16)] = rows_st[pl.ds(src, 16)]
                pltpu.make_async_copy(xt_hbm.at[idxvs[par]], bufs[par],
                                      sems[par]).wait()
                scale_chunk(g, bufs[par])
                pltpu.async_copy(bufs[par], acc.at[rowvs[par]], sem_s,
                                 add=True)

            for q in range(PD - 1):
                build_idx(q, idxvs[q])
                pltpu.async_copy(xt_hbm.at[idxvs[q]], bufs[q], sems[q])

            def outer(g4, c2):
                for par in range(PD):
                    process(g4 * PD + par, par)
                return c2

            lax.fori_loop(0, (NCHUNK - 2) // PD, outer, 0)
            for q in range(((NCHUNK - 2) // PD) * PD, NCHUNK):
                process(jnp.int32(q), q % PD)

            for q in range(NCHUNK - PD, NCHUNK):
                pltpu.make_async_copy(bufs[q % PD], acc.at[rowvs[q % PD]],
                                      sem_s).wait()
            plsc.subcore_barrier()

            def out_chunk(rc, c2):
                ro = r0 + rc * RC
                pltpu.sync_copy(acc.at[pl.ds(ro, RC)], buf0)
                pltpu.sync_copy(buf0,
                                out_hbm.at[b, pl.ds(ro, RC), pl.ds(ho, H)])
                return c2

            lax.fori_loop(0, nrch, out_chunk, 0)
            return carry

        lax.fori_loop(0, B, per_graph, 0)

    return gin_sc


_GIN_SC = _build_sc_call()


@jax.jit
def _gin(inputs, edge_index, adj_values, epsilon):
    ei = edge_index.astype(jnp.int32)
    rows = ei[:, :, 0, :].reshape(B * C * E)
    cols = ei[:, :, 1, :].reshape(B * C * E)
    vals = adj_values.astype(jnp.float32).reshape(B * C * E)
    xt = (inputs.reshape(B, N, 2, H)
          .transpose(0, 2, 1, 3)
          .reshape(B * 2 * N, H))
    esv = jnp.full((16,), jnp.sum(epsilon), jnp.float32)
    return _GIN_SC(xt, rows, cols, vals, esv)


def kernel(inputs, edge_index, adj_values, epsilon):
    return _gin(inputs, edge_index, adj_values, epsilon)
